# baseline (device time: 24342 ns/iter reference)
import jax
import jax.numpy as jnp
from jax import lax
from jax.experimental import pallas as pl
from jax.experimental.pallas import tpu as pltpu

N_DEV = 4
NBLK = 8


def _combine(bv, bi, cv, ci):
    take = (cv > bv) | ((cv == bv) & (ci < bi))
    return jnp.where(take, cv, bv), jnp.where(take, ci, bi)


def kernel(x):
    m_per, n = x.shape
    blk = n // NBLK

    def body(x_hbm, out_ref, xbuf, comm, cpy_sems, s1s, s1r, s2s, s2r):
        my_pos = lax.axis_index("i")
        p1 = my_pos ^ 1
        p2 = my_pos ^ 3

        def copy_in(b):
            return pltpu.make_async_copy(
                x_hbm.at[:, pl.ds(b * blk, blk)],
                xbuf.at[b % 2],
                cpy_sems.at[b % 2],
            )

        def rdma1(b):
            return pltpu.make_async_remote_copy(
                src_ref=comm.at[0, b],
                dst_ref=comm.at[1, b],
                send_sem=s1s.at[b],
                recv_sem=s1r.at[b],
                device_id=(p1,),
                device_id_type=pl.DeviceIdType.MESH,
            )

        def rdma2(b):
            return pltpu.make_async_remote_copy(
                src_ref=comm.at[2, b],
                dst_ref=comm.at[3, b],
                send_sem=s2s.at[b],
                recv_sem=s2r.at[b],
                device_id=(p2,),
                device_id_type=pl.DeviceIdType.MESH,
            )

        copy_in(0).start()

        barrier_sem = pltpu.get_barrier_semaphore()
        for nbr in [p1, p2]:
            pl.semaphore_signal(
                barrier_sem, inc=1,
                device_id=(nbr,), device_id_type=pl.DeviceIdType.MESH,
            )
        pl.semaphore_wait(barrier_sem, 2)

        row_iota = lax.broadcasted_iota(jnp.int32, (m_per, blk), 0)
        pos_f = my_pos.astype(jnp.float32) * jnp.float32(m_per)

        def s1_done(b):
            rdma1(b).wait()
            bv, bi = _combine(
                comm[0, b, 0, :], comm[0, b, 1, :],
                comm[1, b, 0, :], comm[1, b, 1, :],
            )
            comm[2, b, 0, :] = bv
            comm[2, b, 1, :] = bi
            rdma2(b).start()

        def s2_done(b):
            rdma2(b).wait()
            bv, bi = _combine(
                comm[2, b, 0, :], comm[2, b, 1, :],
                comm[3, b, 0, :], comm[3, b, 1, :],
            )
            out_ref[0, pl.ds(b * blk, blk)] = bv
            out_ref[1, pl.ds(b * blk, blk)] = bi

        for b in range(NBLK):
            if b + 1 < NBLK:
                copy_in(b + 1).start()
            copy_in(b).wait()
            xv = xbuf[b % 2]
            val = jnp.max(xv, axis=0)
            idx_local = jnp.min(
                jnp.where(xv == val[None, :], row_iota, jnp.int32(2 * m_per)),
                axis=0,
            )
            idx = idx_local.astype(jnp.float32) + pos_f
            comm[0, b, 0, :] = val
            comm[0, b, 1, :] = idx
            rdma1(b).start()
            if b >= 1:
                s1_done(b - 1)
            if b >= 2:
                s2_done(b - 2)

        s1_done(NBLK - 1)
        s2_done(NBLK - 2)
        s2_done(NBLK - 1)

    return pl.pallas_call(
        body,
        out_shape=jax.ShapeDtypeStruct((2, n), jnp.float32),
        in_specs=[pl.BlockSpec(memory_space=pl.ANY)],
        out_specs=pl.BlockSpec(memory_space=pltpu.VMEM),
        scratch_shapes=[
            pltpu.VMEM((2, m_per, blk), jnp.float32),
            pltpu.VMEM((4, NBLK, 2, blk), jnp.float32),
            pltpu.SemaphoreType.DMA((2,)),
            pltpu.SemaphoreType.DMA((NBLK,)),
            pltpu.SemaphoreType.DMA((NBLK,)),
            pltpu.SemaphoreType.DMA((NBLK,)),
            pltpu.SemaphoreType.DMA((NBLK,)),
        ],
        compiler_params=pltpu.CompilerParams(collective_id=0),
    )(x)


# device time: 17175 ns/iter; 1.4173x vs baseline; 1.4173x over previous
import jax
import jax.numpy as jnp
from jax import lax
from jax.experimental import pallas as pl
from jax.experimental.pallas import tpu as pltpu

N_DEV = 4
NBLK = 8


def _combine(bv, bi, cv, ci):
    take = (cv > bv) | ((cv == bv) & (ci < bi))
    return jnp.where(take, cv, bv), jnp.where(take, ci, bi)


def kernel(x):
    m_per, n = x.shape
    rblk = m_per // NBLK

    def body(x_hbm, out_ref, xbuf, comm, cpy_sems, send_sems, recv_sems):
        my_pos = lax.axis_index("i")
        p1 = my_pos ^ 1
        p2 = my_pos ^ 3

        def copy_in(b):
            return pltpu.make_async_copy(
                x_hbm.at[pl.ds(b * rblk, rblk), :],
                xbuf.at[b % 2],
                cpy_sems.at[b % 2],
            )

        copy_in(0).start()

        barrier_sem = pltpu.get_barrier_semaphore()
        for nbr in [p1, p2]:
            pl.semaphore_signal(
                barrier_sem, inc=1,
                device_id=(nbr,), device_id_type=pl.DeviceIdType.MESH,
            )
        pl.semaphore_wait(barrier_sem, 2)

        row_iota = lax.broadcasted_iota(jnp.int32, (rblk, n), 0)
        big = jnp.int32(2 * m_per)

        best_v = None
        best_i = None
        for b in range(NBLK):
            if b + 1 < NBLK:
                copy_in(b + 1).start()
            copy_in(b).wait()
            xv = xbuf[b % 2]
            mb = jnp.max(xv, axis=0)
            ib = jnp.min(jnp.where(xv == mb[None, :], row_iota, big), axis=0)
            ib = ib + jnp.int32(b * rblk)
            if best_v is None:
                best_v, best_i = mb, ib
            else:
                best_v, best_i = _combine(best_v, best_i, mb, ib)

        val = best_v
        idx = best_i.astype(jnp.float32) + my_pos.astype(jnp.float32) * jnp.float32(
            m_per
        )

        comm[0, 0, :] = val
        comm[0, 1, :] = idx

        r1 = pltpu.make_async_remote_copy(
            src_ref=comm.at[0],
            dst_ref=comm.at[1],
            send_sem=send_sems.at[0],
            recv_sem=recv_sems.at[0],
            device_id=(p1,),
            device_id_type=pl.DeviceIdType.MESH,
        )
        r1.start()
        r1.wait()
        bv, bi = _combine(val, idx, comm[1, 0, :], comm[1, 1, :])

        comm[2, 0, :] = bv
        comm[2, 1, :] = bi
        r2 = pltpu.make_async_remote_copy(
            src_ref=comm.at[2],
            dst_ref=comm.at[3],
            send_sem=send_sems.at[1],
            recv_sem=recv_sems.at[1],
            device_id=(p2,),
            device_id_type=pl.DeviceIdType.MESH,
        )
        r2.start()
        r2.wait()
        bv, bi = _combine(bv, bi, comm[3, 0, :], comm[3, 1, :])

        out_ref[0, :] = bv
        out_ref[1, :] = bi

    return pl.pallas_call(
        body,
        out_shape=jax.ShapeDtypeStruct((2, n), jnp.float32),
        in_specs=[pl.BlockSpec(memory_space=pl.ANY)],
        out_specs=pl.BlockSpec(memory_space=pltpu.VMEM),
        scratch_shapes=[
            pltpu.VMEM((2, rblk, n), jnp.float32),
            pltpu.VMEM((4, 2, n), jnp.float32),
            pltpu.SemaphoreType.DMA((2,)),
            pltpu.SemaphoreType.DMA((2,)),
            pltpu.SemaphoreType.DMA((2,)),
        ],
        compiler_params=pltpu.CompilerParams(collective_id=0),
    )(x)


# device time: 15125 ns/iter; 1.6094x vs baseline; 1.1355x over previous
import jax
import jax.numpy as jnp
from jax import lax
from jax.experimental import pallas as pl
from jax.experimental.pallas import tpu as pltpu

N_DEV = 4
NBLK = 8


def _combine(bv, bi, cv, ci):
    take = (cv > bv) | ((cv == bv) & (ci < bi))
    return jnp.where(take, cv, bv), jnp.where(take, ci, bi)


def kernel(x):
    m_per, n = x.shape
    rblk = m_per // NBLK

    def body(x_ref, out_ref, acc, comm, send_sems, recv_sems):
        k = pl.program_id(0)
        my_pos = lax.axis_index("i")
        p1 = my_pos ^ 1
        p2 = my_pos ^ 3
        diag = my_pos ^ 2

        @pl.when(k == 0)
        def _():
            barrier_sem = pltpu.get_barrier_semaphore()
            for nbr in [p1, p2, diag]:
                pl.semaphore_signal(
                    barrier_sem, inc=1,
                    device_id=(nbr,), device_id_type=pl.DeviceIdType.MESH,
                )
            pl.semaphore_wait(barrier_sem, 3)

        xv = x_ref[:, :]
        mb = jnp.max(xv, axis=0)
        row_iota = lax.broadcasted_iota(jnp.int32, (rblk, n), 0)
        ib = jnp.min(jnp.where(xv == mb[None, :], row_iota, jnp.int32(2 * m_per)), axis=0)
        ib = (ib + k * rblk).astype(jnp.float32)

        @pl.when(k == 0)
        def _():
            acc[0, :] = mb
            acc[1, :] = ib

        @pl.when(k > 0)
        def _():
            bv, bi = _combine(acc[0, :], acc[1, :], mb, ib)
            acc[0, :] = bv
            acc[1, :] = bi

        @pl.when(k == NBLK - 1)
        def _():
            comm[0, 0, :] = acc[0, :]
            comm[0, 1, :] = acc[1, :] + my_pos.astype(jnp.float32) * jnp.float32(m_per)

            rdmas = []
            for s, tgt in ((1, p1), (2, p2), (3, diag)):
                r = pltpu.make_async_remote_copy(
                    src_ref=comm.at[0],
                    dst_ref=comm.at[s],
                    send_sem=send_sems.at[s - 1],
                    recv_sem=recv_sems.at[s - 1],
                    device_id=(tgt,),
                    device_id_type=pl.DeviceIdType.MESH,
                )
                r.start()
                rdmas.append(r)

            bv = comm[0, 0, :]
            bi = comm[0, 1, :]
            for s, r in enumerate(rdmas, start=1):
                r.wait()
                bv, bi = _combine(bv, bi, comm[s, 0, :], comm[s, 1, :])

            out_ref[0, :] = bv
            out_ref[1, :] = bi

    return pl.pallas_call(
        body,
        grid=(NBLK,),
        out_shape=jax.ShapeDtypeStruct((2, n), jnp.float32),
        in_specs=[pl.BlockSpec((rblk, n), lambda k: (k, 0))],
        out_specs=pl.BlockSpec((2, n), lambda k: (0, 0)),
        scratch_shapes=[
            pltpu.VMEM((2, n), jnp.float32),
            pltpu.VMEM((4, 2, n), jnp.float32),
            pltpu.SemaphoreType.DMA((3,)),
            pltpu.SemaphoreType.DMA((3,)),
        ],
        compiler_params=pltpu.CompilerParams(collective_id=0),
    )(x)
